# baseline (speedup 1.0000x reference)
import functools

import jax
import jax.numpy as jnp
from jax import lax
from jax.experimental import pallas as pl
from jax.experimental.pallas import tpu as pltpu
from jax.experimental.pallas import tpu_sc as plsc

NC = 2
NS = 16
NW = NC * NS
K = 128
D = 128

_N = 10000
_E = 320000
_NB = 80
_EPAD = NW * _NB * K
_HN = _N // NC
_ACC_R = 5008
_AR = _ACC_R // NS
_NPAD = 10240
_ZR = _NPAD // NS
_RB = 1000


def _sc_mesh():
    return plsc.VectorSubcoreMesh(
        core_axis_name="c", subcore_axis_name="s", num_cores=NC, num_subcores=NS)


def _sc_degree_inv(dst3, ones_k, zeros1):
    @functools.partial(
        pl.kernel,
        out_type=jax.ShapeDtypeStruct((_NPAD,), jnp.float32),
        mesh=_sc_mesh(),
        scratch_types=[
            pltpu.VMEM((2, _NB, K), jnp.int32),
            pltpu.VMEM((K,), jnp.float32),
            pltpu.VMEM((_ZR,), jnp.float32),
            pltpu.VMEM((_ZR,), jnp.float32),
            pltpu.VMEM_SHARED((_NPAD,), jnp.float32),
        ],
    )
    def body(dst_hbm, ones_hbm, z1_hbm, inv_hbm, dloc, ones_v, cbuf, ibuf, cnt_sh):
        c = lax.axis_index("c")
        s = lax.axis_index("s")
        pltpu.sync_copy(z1_hbm.at[pl.ds(s * _ZR, _ZR)], cnt_sh.at[pl.ds(s * _ZR, _ZR)])
        pltpu.sync_copy(dst_hbm.at[pl.ds(2 * s, 2)], dloc)
        pltpu.sync_copy(ones_hbm, ones_v)
        plsc.subcore_barrier()

        @pl.loop(0, 2 * _NB)
        def _(i):
            pltpu.sync_copy(ones_v, cnt_sh.at[dloc.at[i // _NB, i % _NB]], add=True)

        plsc.subcore_barrier()
        pltpu.sync_copy(cnt_sh.at[pl.ds(s * _ZR, _ZR)], cbuf)
        for k in range(_ZR // 16):
            v = cbuf[pl.ds(k * 16, 16)]
            ibuf[pl.ds(k * 16, 16)] = 1.0 / jnp.maximum(v, 1.0)

        @pl.when(c == 0)
        def _():
            pltpu.sync_copy(ibuf, inv_hbm.at[pl.ds(s * _ZR, _ZR)])

    return body(dst3, ones_k, zeros1)


def _sc_segment_sum(y, src3, dst3):
    M = 4
    DPF = 2

    @functools.partial(
        pl.kernel,
        out_type=jax.ShapeDtypeStruct((NC, _ACC_R, D), jnp.float32),
        mesh=_sc_mesh(),
        scratch_types=[
            pltpu.VMEM((_NB, K), jnp.int32),
            pltpu.VMEM((_NB, K), jnp.int32),
            pltpu.VMEM((M, K, D), jnp.float32),
            pltpu.VMEM_SHARED((_ACC_R, D), jnp.float32),
            pltpu.SemaphoreType.DMA((M,)),
            pltpu.SemaphoreType.DMA((M,)),
        ],
        compiler_params=pltpu.CompilerParams(use_tc_tiling_on_sc=False),
    )
    def body(y_hbm, src_hbm, dst_hbm, out_hbm, sloc, dloc, rows, acc, gsem, ssem):
        c = lax.axis_index("c")
        s = lax.axis_index("s")
        lo = c * _HN
        lov = jnp.zeros((16,), jnp.int32) + lo

        zv = jnp.zeros((16,), jnp.float32)
        for r in range(K):
            for q in range(D // 16):
                rows[0, r, pl.ds(q * 16, 16)] = zv
        for t in range(_AR // K):
            pltpu.sync_copy(rows.at[0], acc.at[pl.ds(s * _AR + t * K, K)])
        rem = _AR % K
        pltpu.sync_copy(rows.at[0, pl.ds(0, rem)],
                        acc.at[pl.ds(s * _AR + (_AR // K) * K, rem)])
        plsc.subcore_barrier()

        for jc in range(2):
            pltpu.sync_copy(src_hbm.at[2 * s + jc], sloc)
            pltpu.sync_copy(dst_hbm.at[2 * s + jc], dloc)

            @pl.loop(0, _NB)
            def _(bb):
                for q in range(K // 16):
                    v = dloc[bb, pl.ds(q * 16, 16)] - lov
                    ok = (v >= 0) & (v < _HN)
                    dloc[bb, pl.ds(q * 16, 16)] = jnp.where(ok, v, _HN)

            for j in range(DPF):
                pltpu.async_copy(y_hbm.at[sloc.at[j]], rows.at[j], gsem.at[j])

            @pl.loop(0, _NB, step=M)
            def _(g):
                for j in range(M):
                    b = g + j
                    pltpu.make_async_copy(y_hbm.at[sloc.at[b]], rows.at[j],
                                          gsem.at[j]).wait()
                    pltpu.async_copy(rows.at[j], acc.at[dloc.at[b]], ssem.at[j],
                                     add=True)
                    k = (j + DPF) % M

                    @pl.when(b + DPF < _NB)
                    def _():
                        @pl.when(b + DPF >= M)
                        def _():
                            pltpu.make_async_copy(rows.at[k], acc.at[dloc.at[0]],
                                                  ssem.at[k]).wait()
                        pltpu.async_copy(y_hbm.at[sloc.at[b + DPF]],
                                         rows.at[k], gsem.at[k])

            for j in range(M):
                pltpu.make_async_copy(rows.at[j], acc.at[dloc.at[0]],
                                      ssem.at[j]).wait()

        plsc.subcore_barrier()
        pltpu.sync_copy(acc.at[pl.ds(s * _AR, _AR)],
                        out_hbm.at[c, pl.ds(s * _AR, _AR)])

    return body(y, src3, dst3)


def _vspec(d):
    return pl.BlockSpec((1, d), lambda i: (0, 0))


def _row(d):
    return pl.BlockSpec((_RB, d), lambda i: (i, 0))


def _pspec():
    return pl.BlockSpec((1, _RB, D), lambda i: (i // 5, i % 5, 0))


def _tc_pre(x, wlt, g, rv):
    din, do = wlt.shape

    def body(x_ref, w_ref, g_ref, rv_ref, o_ref):
        sc = g_ref[...] * lax.rsqrt(rv_ref[...] + 1e-5)
        o_ref[...] = jnp.dot(x_ref[...], w_ref[...] * sc,
                             preferred_element_type=jnp.float32)

    return pl.pallas_call(
        body,
        grid=(_N // _RB,),
        in_specs=[_row(din), pl.BlockSpec((din, do), lambda i: (0, 0)),
                  _vspec(do), _vspec(do)],
        out_specs=_row(do),
        out_shape=jax.ShapeDtypeStruct((_N, do), jnp.float32),
    )(x, wlt, g.reshape(1, -1), rv.reshape(1, -1))


def _tc_mid(p, inv, xin, wrt, bl, g, b, rm, rv, res, wnt, gn, rvn):
    din, do = wrt.shape
    dn = wnt.shape[1]
    has_res = res is not None

    def body(*refs):
        pr, ivr, xr, wr, blr, gr, br, rmr, rvr = refs[:9]
        i = 9
        if has_res:
            resr = refs[i]
            i += 1
        wnr, gnr, rvnr, hr, ynr = refs[i:i + 5]
        sc = gr[...] * lax.rsqrt(rvr[...] + 1e-5)
        m = pr[0] * ivr[...]
        pre = (m + jnp.dot(xr[...], wr[...] * sc, preferred_element_type=jnp.float32)
               + (blr[...] - rmr[...]) * sc + br[...])
        h_out = jnp.maximum(pre, 0.0)
        if has_res:
            h_out = h_out + resr[...]
        hr[...] = h_out
        scn = gnr[...] * lax.rsqrt(rvnr[...] + 1e-5)
        yn = jnp.dot(h_out, wnr[...] * scn, preferred_element_type=jnp.float32)
        if dn < D:
            yn = jnp.concatenate([yn, yn], axis=1)
        ynr[...] = yn

    in_specs = [_pspec(), pl.BlockSpec((_RB, 1), lambda i: (i, 0)),
                _row(din), pl.BlockSpec((din, do), lambda i: (0, 0)),
                _vspec(do), _vspec(do), _vspec(do), _vspec(do), _vspec(do)]
    args = [p, inv, xin, wrt, bl.reshape(1, -1), g.reshape(1, -1),
            b.reshape(1, -1), rm.reshape(1, -1), rv.reshape(1, -1)]
    if has_res:
        in_specs.append(_row(do))
        args.append(res)
    in_specs += [pl.BlockSpec((do, dn), lambda i: (0, 0)), _vspec(dn), _vspec(dn)]
    args += [wnt, gn.reshape(1, -1), rvn.reshape(1, -1)]

    return pl.pallas_call(
        body,
        grid=(_N // _RB,),
        in_specs=in_specs,
        out_specs=(_row(do), _row(D)),
        out_shape=(jax.ShapeDtypeStruct((_N, do), jnp.float32),
                   jax.ShapeDtypeStruct((_N, D), jnp.float32)),
    )(*args)


def _tc_fin(p, inv, xin, wrt, bl, g, b, rm, rv, wot, bo):
    din, do = wrt.shape

    def body(pr, ivr, xr, wr, blr, gr, br, rmr, rvr, wor, bor, or_):
        sc = gr[...] * lax.rsqrt(rvr[...] + 1e-5)
        m = pr[0][:, :do] * ivr[...]
        pre = (m + jnp.dot(xr[...], wr[...] * sc, preferred_element_type=jnp.float32)
               + (blr[...] - rmr[...]) * sc + br[...])
        h = jnp.maximum(pre, 0.0)
        or_[...] = jnp.dot(h, wor[...], preferred_element_type=jnp.float32) + bor[0, 0]

    return pl.pallas_call(
        body,
        grid=(_N // _RB,),
        in_specs=[_pspec(), pl.BlockSpec((_RB, 1), lambda i: (i, 0)),
                  _row(din), pl.BlockSpec((din, do), lambda i: (0, 0)),
                  _vspec(do), _vspec(do), _vspec(do), _vspec(do), _vspec(do),
                  pl.BlockSpec((do, 1), lambda i: (0, 0)),
                  pl.BlockSpec(memory_space=pltpu.MemorySpace.SMEM)],
        out_specs=pl.BlockSpec((_RB, 1), lambda i: (i, 0)),
        out_shape=jax.ShapeDtypeStruct((_N, 1), jnp.float32),
    )(p, inv, xin, wrt,
      bl.reshape(1, -1), g.reshape(1, -1), b.reshape(1, -1),
      rm.reshape(1, -1), rv.reshape(1, -1), wot, bo.reshape(1, 1))


def kernel(x, edge_index, Wl1, bl1, Wr1, g1, b1, rm1, rv1,
           Wl2, bl2, Wr2, g2, b2, rm2, rv2,
           Wl3, bl3, Wr3, g3, b3, rm3, rv3, Wo, bo):
    src = edge_index[0]
    dst = edge_index[1]
    pad = _EPAD - _E
    src3 = jnp.concatenate([src, jnp.zeros((pad,), src.dtype)]).reshape(NW, _NB, K)
    dst3 = jnp.concatenate([dst, jnp.full((pad,), _N, dst.dtype)]).reshape(NW, _NB, K)

    zeros1 = jnp.zeros((_NPAD,), jnp.float32)
    ones_k = jnp.ones((K,), jnp.float32)

    inv = _sc_degree_inv(dst3, ones_k, zeros1)
    inv_col = inv[:_N].reshape(_N, 1)

    y1 = _tc_pre(x, Wl1.T, g1, rv1)
    p1 = _sc_segment_sum(y1, src3, dst3)
    h1, y2 = _tc_mid(p1, inv_col, x, Wr1.T, bl1, g1, b1, rm1, rv1,
                     None, Wl2.T, g2, rv2)
    p2 = _sc_segment_sum(y2, src3, dst3)
    h2, y3 = _tc_mid(p2, inv_col, h1, Wr2.T, bl2, g2, b2, rm2, rv2,
                     h1, Wl3.T, g3, rv3)
    p3 = _sc_segment_sum(y3, src3, dst3)
    out = _tc_fin(p3, inv_col, h2, Wr3.T, bl3, g3, b3, rm3, rv3, Wo.T, bo)
    return out.reshape(_N)


# speedup vs baseline: 1.7871x; 1.7871x over previous
import functools

import jax
import jax.numpy as jnp
from jax import lax
from jax.experimental import pallas as pl
from jax.experimental.pallas import tpu as pltpu
from jax.experimental.pallas import tpu_sc as plsc

NC = 2
NS = 16
NW = NC * NS
K = 128
DH = 64

_N = 10000
_E = 320000
_NB = 80
_EPAD = NW * _NB * K
_NPAD = 10240
_ZR = _NPAD // NS
_RB = 2000


def _sc_mesh():
    return plsc.VectorSubcoreMesh(
        core_axis_name="c", subcore_axis_name="s", num_cores=NC, num_subcores=NS)


def _sc_degree_inv(dst3, ones_k, zeros1):
    @functools.partial(
        pl.kernel,
        out_type=jax.ShapeDtypeStruct((_NPAD,), jnp.float32),
        mesh=_sc_mesh(),
        scratch_types=[
            pltpu.VMEM((2, _NB, K), jnp.int32),
            pltpu.VMEM((K,), jnp.float32),
            pltpu.VMEM((_ZR,), jnp.float32),
            pltpu.VMEM((_ZR,), jnp.float32),
            pltpu.VMEM_SHARED((_NPAD,), jnp.float32),
        ],
    )
    def body(dst_hbm, ones_hbm, z1_hbm, inv_hbm, dloc, ones_v, cbuf, ibuf, cnt_sh):
        c = lax.axis_index("c")
        s = lax.axis_index("s")
        pltpu.sync_copy(z1_hbm.at[pl.ds(s * _ZR, _ZR)], cnt_sh.at[pl.ds(s * _ZR, _ZR)])
        pltpu.sync_copy(dst_hbm.at[pl.ds(2 * s, 2)], dloc)
        pltpu.sync_copy(ones_hbm, ones_v)
        plsc.subcore_barrier()

        @pl.loop(0, 2 * _NB)
        def _(i):
            pltpu.sync_copy(ones_v, cnt_sh.at[dloc.at[i // _NB, i % _NB]], add=True)

        plsc.subcore_barrier()
        pltpu.sync_copy(cnt_sh.at[pl.ds(s * _ZR, _ZR)], cbuf)
        for k in range(_ZR // 16):
            v = cbuf[pl.ds(k * 16, 16)]
            ibuf[pl.ds(k * 16, 16)] = 1.0 / jnp.maximum(v, 1.0)

        @pl.when(c == 0)
        def _():
            pltpu.sync_copy(ibuf, inv_hbm.at[pl.ds(s * _ZR, _ZR)])

    return body(dst3, ones_k, zeros1)


def _sc_segment_sum(y_halves, src3, dst3, zeros2):
    nh = len(y_halves)

    @functools.partial(
        pl.kernel,
        out_type=tuple(jax.ShapeDtypeStruct((NC, _NPAD, DH), jnp.float32)
                       for _ in range(nh)),
        mesh=_sc_mesh(),
        scratch_types=[
            pltpu.VMEM((_NB, K), jnp.int32),
            pltpu.VMEM((_NB, K), jnp.int32),
            pltpu.VMEM((2, K, DH), jnp.float32),
            pltpu.VMEM_SHARED((_NPAD, DH), jnp.float32),
            pltpu.SemaphoreType.DMA,
            pltpu.SemaphoreType.DMA,
        ],
        compiler_params=pltpu.CompilerParams(use_tc_tiling_on_sc=False),
    )
    def body(*refs):
        y_refs = refs[:nh]
        src_hbm, dst_hbm, z2_hbm = refs[nh:nh + 3]
        out_refs = refs[nh + 3:2 * nh + 3]
        sloc, dloc, rows, acc, sem0, sem1 = refs[2 * nh + 3:]
        c = lax.axis_index("c")
        s = lax.axis_index("s")
        wid = c * NS + s
        pltpu.sync_copy(src_hbm.at[wid], sloc)
        pltpu.sync_copy(dst_hbm.at[wid], dloc)

        for y_hbm, out_hbm in zip(y_refs, out_refs):
            pltpu.sync_copy(z2_hbm.at[pl.ds(s * _ZR, _ZR)], acc.at[pl.ds(s * _ZR, _ZR)])
            plsc.subcore_barrier()

            pltpu.async_copy(y_hbm.at[sloc.at[0]], rows.at[0], sem0)

            @pl.loop(0, _NB, step=2)
            def _(g):
                pltpu.async_copy(y_hbm.at[sloc.at[g + 1]], rows.at[1], sem1)
                pltpu.make_async_copy(y_hbm.at[sloc.at[g]], rows.at[0], sem0).wait()
                pltpu.sync_copy(rows.at[0], acc.at[dloc.at[g]], add=True)

                @pl.when(g + 2 < _NB)
                def _():
                    pltpu.async_copy(y_hbm.at[sloc.at[g + 2]], rows.at[0], sem0)

                pltpu.make_async_copy(y_hbm.at[sloc.at[g + 1]], rows.at[1], sem1).wait()
                pltpu.sync_copy(rows.at[1], acc.at[dloc.at[g + 1]], add=True)

            plsc.subcore_barrier()
            pltpu.sync_copy(acc.at[pl.ds(s * _ZR, _ZR)],
                            out_hbm.at[c, pl.ds(s * _ZR, _ZR)])

    out = body(*y_halves, src3, dst3, zeros2)
    return list(out) if isinstance(out, (tuple, list)) else [out]


def _vspec(d):
    return pl.BlockSpec((1, d), lambda i: (0, 0))


def _row(d):
    return pl.BlockSpec((_RB, d), lambda i: (i, 0))


def _tc_pre(x, wlt, g, rv):
    din, do = wlt.shape
    nh = do // DH

    def body(x_ref, w_ref, g_ref, rv_ref, *o_refs):
        sc = g_ref[...] * lax.rsqrt(rv_ref[...] + 1e-5)
        y = jnp.dot(x_ref[...], w_ref[...] * sc, preferred_element_type=jnp.float32)
        for h, o_ref in enumerate(o_refs):
            o_ref[...] = y[:, h * DH:(h + 1) * DH]

    return pl.pallas_call(
        body,
        grid=(_N // _RB,),
        in_specs=[_row(din), pl.BlockSpec((din, do), lambda i: (0, 0)),
                  _vspec(do), _vspec(do)],
        out_specs=tuple(_row(DH) for _ in range(nh)),
        out_shape=tuple(jax.ShapeDtypeStruct((_N, DH), jnp.float32)
                        for _ in range(nh)),
    )(x, wlt, g.reshape(1, -1), rv.reshape(1, -1))


def _tc_mid(agg_pairs, inv, xin, wrt, bl, g, b, rm, rv, res, wnt, gn, rvn):
    din, do = wrt.shape
    dn = wnt.shape[1]
    nh = len(agg_pairs)
    nyn = dn // DH
    has_res = res is not None

    def body(*refs):
        a_refs = refs[:2 * nh]
        i = 2 * nh
        ivr, xr, wr, blr, gr, br, rmr, rvr = refs[i:i + 8]
        i += 8
        if has_res:
            resr = refs[i]
            i += 1
        wnr, gnr, rvnr = refs[i:i + 3]
        hr = refs[i + 3]
        yn_refs = refs[i + 4:]
        sc = gr[...] * lax.rsqrt(rvr[...] + 1e-5)
        mh = [(a_refs[2 * h][...] + a_refs[2 * h + 1][...]) * ivr[...]
              for h in range(nh)]
        m = mh[0] if nh == 1 else jnp.concatenate(mh, axis=1)
        pre = (m + jnp.dot(xr[...], wr[...] * sc, preferred_element_type=jnp.float32)
               + (blr[...] - rmr[...]) * sc + br[...])
        h_out = jnp.maximum(pre, 0.0)
        if has_res:
            h_out = h_out + resr[...]
        hr[...] = h_out
        scn = gnr[...] * lax.rsqrt(rvnr[...] + 1e-5)
        yn = jnp.dot(h_out, wnr[...] * scn, preferred_element_type=jnp.float32)
        for h, yn_ref in enumerate(yn_refs):
            yn_ref[...] = yn[:, h * DH:(h + 1) * DH]

    in_specs = [_row(DH)] * (2 * nh)
    args = [p for pair in agg_pairs for p in pair]
    in_specs += [pl.BlockSpec((_RB, 1), lambda i: (i, 0)), _row(din),
                 pl.BlockSpec((din, do), lambda i: (0, 0)),
                 _vspec(do), _vspec(do), _vspec(do), _vspec(do), _vspec(do)]
    args += [inv, xin, wrt, bl.reshape(1, -1), g.reshape(1, -1),
             b.reshape(1, -1), rm.reshape(1, -1), rv.reshape(1, -1)]
    if has_res:
        in_specs.append(_row(do))
        args.append(res)
    in_specs += [pl.BlockSpec((do, dn), lambda i: (0, 0)), _vspec(dn), _vspec(dn)]
    args += [wnt, gn.reshape(1, -1), rvn.reshape(1, -1)]

    outs = pl.pallas_call(
        body,
        grid=(_N // _RB,),
        in_specs=in_specs,
        out_specs=(_row(do),) + tuple(_row(DH) for _ in range(nyn)),
        out_shape=((jax.ShapeDtypeStruct((_N, do), jnp.float32),)
                   + tuple(jax.ShapeDtypeStruct((_N, DH), jnp.float32)
                           for _ in range(nyn))),
    )(*args)
    return outs[0], list(outs[1:])


def _tc_fin(a0, a1, inv, xin, wrt, bl, g, b, rm, rv, wot, bo):
    din, do = wrt.shape

    def body(a0r, a1r, ivr, xr, wr, blr, gr, br, rmr, rvr, wor, bor, or_):
        sc = gr[...] * lax.rsqrt(rvr[...] + 1e-5)
        m = (a0r[...] + a1r[...]) * ivr[...]
        pre = (m + jnp.dot(xr[...], wr[...] * sc, preferred_element_type=jnp.float32)
               + (blr[...] - rmr[...]) * sc + br[...])
        h = jnp.maximum(pre, 0.0)
        or_[...] = jnp.dot(h, wor[...], preferred_element_type=jnp.float32) + bor[0, 0]

    return pl.pallas_call(
        body,
        grid=(_N // _RB,),
        in_specs=[_row(do), _row(do), pl.BlockSpec((_RB, 1), lambda i: (i, 0)),
                  _row(din), pl.BlockSpec((din, do), lambda i: (0, 0)),
                  _vspec(do), _vspec(do), _vspec(do), _vspec(do), _vspec(do),
                  pl.BlockSpec((do, 1), lambda i: (0, 0)),
                  pl.BlockSpec(memory_space=pltpu.MemorySpace.SMEM)],
        out_specs=pl.BlockSpec((_RB, 1), lambda i: (i, 0)),
        out_shape=jax.ShapeDtypeStruct((_N, 1), jnp.float32),
    )(a0, a1, inv, xin, wrt,
      bl.reshape(1, -1), g.reshape(1, -1), b.reshape(1, -1),
      rm.reshape(1, -1), rv.reshape(1, -1), wot, bo.reshape(1, 1))


def _pairs(partials):
    return [(p[0, :_N], p[1, :_N]) for p in partials]


def kernel(x, edge_index, Wl1, bl1, Wr1, g1, b1, rm1, rv1,
           Wl2, bl2, Wr2, g2, b2, rm2, rv2,
           Wl3, bl3, Wr3, g3, b3, rm3, rv3, Wo, bo):
    src = edge_index[0]
    dst = edge_index[1]
    pad = _EPAD - _E
    src3 = jnp.concatenate([src, jnp.zeros((pad,), src.dtype)]).reshape(NW, _NB, K)
    dst3 = jnp.concatenate([dst, jnp.full((pad,), _N, dst.dtype)]).reshape(NW, _NB, K)

    zeros1 = jnp.zeros((_NPAD,), jnp.float32)
    zeros2 = jnp.zeros((_NPAD, DH), jnp.float32)
    ones_k = jnp.ones((K,), jnp.float32)

    inv = _sc_degree_inv(dst3, ones_k, zeros1)
    inv_col = inv[:_N].reshape(_N, 1)

    y1h = _tc_pre(x, Wl1.T, g1, rv1)
    p1 = _sc_segment_sum(y1h, src3, dst3, zeros2)
    h1, y2h = _tc_mid(_pairs(p1), inv_col, x, Wr1.T, bl1, g1, b1, rm1, rv1,
                      None, Wl2.T, g2, rv2)
    p2 = _sc_segment_sum(y2h, src3, dst3, zeros2)
    h2, y3h = _tc_mid(_pairs(p2), inv_col, h1, Wr2.T, bl2, g2, b2, rm2, rv2,
                      h1, Wl3.T, g3, rv3)
    p3 = _sc_segment_sum(y3h, src3, dst3, zeros2)
    (a0, a1), = _pairs(p3)
    out = _tc_fin(a0, a1, inv_col, h2, Wr3.T, bl3, g3, b3, rm3, rv3, Wo.T, bo)
    return out.reshape(_N)
